# baseline (speedup 1.0000x reference)
import jax
import jax.numpy as jnp
from jax import lax
from jax.experimental import pallas as pl
from jax.experimental.pallas import tpu as pltpu
from jax.experimental.pallas import tpu_sc as plsc

NC = 2
NS = 16
NW = NC * NS
LANES = 16
B = 1024
L = 20
D = 64
NBW = B // NW
H = 4
RPC = H * L
NSL = D // LANES


def _wide_gather(V, W, n_out, four_d, replicated=False):
    rows_per_w = NBW * L * (L if four_d else 1)
    n_chunks = rows_per_w // RPC
    cpb = L // H

    mesh = plsc.VectorSubcoreMesh(core_axis_name="c", subcore_axis_name="s")
    oshape = (B, L, L, D) if four_d else (B, L, D)
    out_type = [jax.ShapeDtypeStruct(oshape, jnp.float32) for _ in range(n_out)]
    scratch = [pltpu.VMEM((rows_per_w,), jnp.int32)]
    scratch += [pltpu.VMEM((RPC, W), jnp.float32) for _ in range(2)]
    scratch += [pltpu.VMEM((H, L, D), jnp.float32) for _ in range(2 * n_out)]
    scratch += [pltpu.SemaphoreType.DMA for _ in range(2 + 2 * n_out)]

    def body(table_hbm, idx_hbm, *refs):
        outs = refs[0:n_out]
        idx_v = refs[n_out]
        wide = refs[n_out + 1:n_out + 3]
        rows = tuple(
            refs[n_out + 3 + 2 * t:n_out + 5 + 2 * t] for t in range(n_out)
        )
        gsems = refs[n_out + 3 + 2 * n_out:n_out + 5 + 2 * n_out]
        osems = tuple(
            refs[n_out + 5 + 2 * n_out + 2 * t:n_out + 7 + 2 * n_out + 2 * t]
            for t in range(n_out)
        )

        wid = lax.axis_index("s") * NC + lax.axis_index("c")
        b0 = wid * NBW

        def out_slice(t, k):
            if four_d:
                return outs[t].at[b0 + k // cpb, pl.ds(lax.rem(k, cpb) * H, H)]
            return outs[t].at[pl.ds(b0 + k * H, H)]

        pltpu.sync_copy(idx_hbm.at[pl.ds(wid * rows_per_w, rows_per_w)], idx_v)

        if replicated:
            voff = wid * V

            def shift_body(q, carry):
                sl = pl.ds(q * LANES, LANES)
                idx_v[sl] = idx_v[sl] + voff
                return carry

            lax.fori_loop(0, rows_per_w // LANES, shift_body, 0)

        pltpu.async_copy(table_hbm.at[idx_v.at[pl.ds(0, RPC)]], wide[0], gsems[0])

        def pair_body(p, carry):
            for i in range(2):
                k = p * 2 + i

                @pl.when(k + 1 < n_chunks)
                def _prefetch():
                    pltpu.async_copy(
                        table_hbm.at[idx_v.at[pl.ds((k + 1) * RPC, RPC)]],
                        wide[1 - i], gsems[1 - i],
                    )

                pltpu.make_async_copy(
                    table_hbm.at[idx_v.at[pl.ds(0, RPC)]], wide[i], gsems[i]
                ).wait()

                @pl.when(p > 0)
                def _wait_out():
                    for t in range(n_out):
                        pltpu.make_async_copy(
                            out_slice(t, 0), rows[t][i], osems[t][i]
                        ).wait()

                for ib in range(H):
                    for ll in range(L):
                        r = ib * L + ll
                        for t in range(n_out):
                            for c in range(NSL):
                                rows[t][i][ib, ll, pl.ds(c * LANES, LANES)] = (
                                    wide[i][r, pl.ds(t * D + c * LANES, LANES)]
                                )

                for t in range(n_out):
                    pltpu.async_copy(rows[t][i], out_slice(t, k), osems[t][i])
            return carry

        lax.fori_loop(0, n_chunks // 2, pair_body, 0)
        for t in range(n_out):
            for i in range(2):
                pltpu.make_async_copy(
                    out_slice(t, 0), rows[t][i], osems[t][i]
                ).wait()

    return pl.kernel(body, out_type=out_type, mesh=mesh, scratch_types=scratch)


NPB = 8
NBB = NW // NPB
PPW = L * L // NPB
BPW = B // NBB


def _time_gather_t():
    mesh = plsc.VectorSubcoreMesh(core_axis_name="c", subcore_axis_name="s")
    out_type = [
        jax.ShapeDtypeStruct((L, L, D, B), jnp.float32) for _ in range(2)
    ]
    scratch = [pltpu.VMEM((256 * 2 * D,), jnp.float32)]
    scratch += [pltpu.VMEM((PPW, BPW), jnp.int32)]
    scratch += [pltpu.VMEM((D, BPW), jnp.float32) for _ in range(4)]
    scratch += [pltpu.SemaphoreType.DMA]
    scratch += [pltpu.SemaphoreType.DMA for _ in range(4)]

    def body(tab_hbm, idx_hbm, ok_hbm, ov_hbm, tab_v, idx_v, *refs):
        rows = (refs[0:2], refs[2:4])
        isem = refs[4]
        osems = (refs[5:7], refs[7:9])
        outs = (ok_hbm, ov_hbm)

        wid = lax.axis_index("s") * NC + lax.axis_index("c")
        pb = wid // NBB
        bb = lax.rem(wid, NBB)
        p0 = pb * PPW
        boff = bb * BPW

        pltpu.sync_copy(tab_hbm, tab_v)
        for q in range(PPW):
            pltpu.async_copy(
                idx_hbm.at[pl.ds((p0 + q) * B + boff, BPW)], idx_v.at[q], isem
            )
        for q in range(PPW):
            pltpu.make_async_copy(
                idx_hbm.at[pl.ds(0, BPW)], idx_v.at[q], isem
            ).wait()

        def pair_pair_body(pp, carry):
            for i in range(2):
                q = pp * 2 + i
                p = p0 + q
                l1 = p // L
                l2 = lax.rem(p, L)

                @pl.when(pp > 0)
                def _wait_out():
                    for t in range(2):
                        pltpu.make_async_copy(
                            outs[t].at[0, 0, pl.ds(0, D), pl.ds(0, BPW)],
                            rows[t][i], osems[t][i],
                        ).wait()

                def group_body(g, c2):
                    iv = idx_v[q, pl.ds(g * LANES, LANES)]
                    fb = iv * (2 * D)
                    for d in range(D):
                        vk = plsc.load_gather(tab_v, [fb + d])
                        rows[0][i][d, pl.ds(g * LANES, LANES)] = vk
                        vv = plsc.load_gather(tab_v, [fb + (D + d)])
                        rows[1][i][d, pl.ds(g * LANES, LANES)] = vv
                    return c2

                lax.fori_loop(0, BPW // LANES, group_body, 0)

                for t in range(2):
                    pltpu.async_copy(
                        rows[t][i],
                        outs[t].at[l1, l2, pl.ds(0, D), pl.ds(boff, BPW)],
                        osems[t][i],
                    )
            return carry

        lax.fori_loop(0, PPW // 2, pair_pair_body, 0)
        for t in range(2):
            for i in range(2):
                pltpu.make_async_copy(
                    outs[t].at[0, 0, pl.ds(0, D), pl.ds(0, BPW)],
                    rows[t][i], osems[t][i],
                ).wait()

    return pl.kernel(
        body, out_type=out_type, mesh=mesh, scratch_types=scratch,
        compiler_params=pltpu.CompilerParams(needs_layout_passes=False))


def kernel(seq_S_u, seq_P_u, T_delta_u, contract_table, time_K_table,
           time_V_table, pop_Q_table, pop_K_table, pop_V_table):
    idx_S = seq_S_u.astype(jnp.int32).reshape(-1)
    idx_Tt = jnp.transpose(T_delta_u.astype(jnp.int32), (1, 2, 0)).reshape(-1)
    idx_P = seq_P_u.astype(jnp.int32).reshape(-1)

    contract_wide = jnp.pad(contract_table, ((0, 0), (0, D)))
    time_flat = jnp.concatenate(
        [time_K_table, time_V_table], axis=1).reshape(-1)
    npop = pop_Q_table.shape[0]
    pop_wide = jnp.tile(
        jnp.concatenate(
            [pop_Q_table, pop_K_table, pop_V_table,
             jnp.zeros((npop, D), jnp.float32)], axis=1), (NW, 1))

    (E,) = _wide_gather(contract_wide.shape[0], 2 * D, 1, False)(
        contract_wide, idx_S)
    T_K_t, T_V_t = _time_gather_t()(time_flat, idx_Tt)
    P_Q, P_K, P_V = _wide_gather(npop, 4 * D, 3, False, replicated=True)(
        pop_wide, idx_P)

    T_K = jnp.transpose(T_K_t, (3, 0, 1, 2))
    T_V = jnp.transpose(T_V_t, (3, 0, 1, 2))
    return (E, T_K, T_V, P_Q, P_K, P_V)


# speedup vs baseline: 2.3551x; 2.3551x over previous
import jax
import jax.numpy as jnp
from jax import lax
from jax.experimental import pallas as pl
from jax.experimental.pallas import tpu as pltpu
from jax.experimental.pallas import tpu_sc as plsc

NC = 2
NS = 16
NW = NC * NS
LANES = 16
B = 1024
L = 20
D = 64
NBW = B // NW
H = 4
RPC = H * L
NSL = D // LANES


def _wide_gather(V, W, n_out, four_d, replicated=False):
    rows_per_w = NBW * L * (L if four_d else 1)
    n_chunks = rows_per_w // RPC
    cpb = L // H

    mesh = plsc.VectorSubcoreMesh(core_axis_name="c", subcore_axis_name="s")
    oshape = (B, L, L, D) if four_d else (B, L, D)
    out_type = [jax.ShapeDtypeStruct(oshape, jnp.float32) for _ in range(n_out)]
    scratch = [pltpu.VMEM((rows_per_w,), jnp.int32)]
    scratch += [pltpu.VMEM((RPC, W), jnp.float32) for _ in range(2)]
    scratch += [pltpu.VMEM((H, L, D), jnp.float32) for _ in range(2 * n_out)]
    scratch += [pltpu.SemaphoreType.DMA for _ in range(2 + 2 * n_out)]

    def body(table_hbm, idx_hbm, *refs):
        outs = refs[0:n_out]
        idx_v = refs[n_out]
        wide = refs[n_out + 1:n_out + 3]
        rows = tuple(
            refs[n_out + 3 + 2 * t:n_out + 5 + 2 * t] for t in range(n_out)
        )
        gsems = refs[n_out + 3 + 2 * n_out:n_out + 5 + 2 * n_out]
        osems = tuple(
            refs[n_out + 5 + 2 * n_out + 2 * t:n_out + 7 + 2 * n_out + 2 * t]
            for t in range(n_out)
        )

        wid = lax.axis_index("s") * NC + lax.axis_index("c")
        b0 = wid * NBW

        def out_slice(t, k):
            if four_d:
                return outs[t].at[b0 + k // cpb, pl.ds(lax.rem(k, cpb) * H, H)]
            return outs[t].at[pl.ds(b0 + k * H, H)]

        pltpu.sync_copy(idx_hbm.at[pl.ds(wid * rows_per_w, rows_per_w)], idx_v)

        if replicated:
            voff = wid * V

            def shift_body(q, carry):
                sl = pl.ds(q * LANES, LANES)
                idx_v[sl] = idx_v[sl] + voff
                return carry

            lax.fori_loop(0, rows_per_w // LANES, shift_body, 0)

        pltpu.async_copy(table_hbm.at[idx_v.at[pl.ds(0, RPC)]], wide[0], gsems[0])

        def pair_body(p, carry):
            for i in range(2):
                k = p * 2 + i

                @pl.when(k + 1 < n_chunks)
                def _prefetch():
                    pltpu.async_copy(
                        table_hbm.at[idx_v.at[pl.ds((k + 1) * RPC, RPC)]],
                        wide[1 - i], gsems[1 - i],
                    )

                pltpu.make_async_copy(
                    table_hbm.at[idx_v.at[pl.ds(0, RPC)]], wide[i], gsems[i]
                ).wait()

                @pl.when(p > 0)
                def _wait_out():
                    for t in range(n_out):
                        pltpu.make_async_copy(
                            out_slice(t, 0), rows[t][i], osems[t][i]
                        ).wait()

                for ib in range(H):
                    for ll in range(L):
                        r = ib * L + ll
                        for t in range(n_out):
                            for c in range(NSL):
                                rows[t][i][ib, ll, pl.ds(c * LANES, LANES)] = (
                                    wide[i][r, pl.ds(t * D + c * LANES, LANES)]
                                )

                for t in range(n_out):
                    pltpu.async_copy(rows[t][i], out_slice(t, k), osems[t][i])
            return carry

        lax.fori_loop(0, n_chunks // 2, pair_body, 0)
        for t in range(n_out):
            for i in range(2):
                pltpu.make_async_copy(
                    out_slice(t, 0), rows[t][i], osems[t][i]
                ).wait()

    return pl.kernel(body, out_type=out_type, mesh=mesh, scratch_types=scratch)


NPB = 8
NBB = NW // NPB
PPW = L * L // NPB
BPW = B // NBB


def _time_gather_t():
    mesh = plsc.VectorSubcoreMesh(core_axis_name="c", subcore_axis_name="s")
    out_type = [
        jax.ShapeDtypeStruct((L, L, D, B), jnp.float32) for _ in range(2)
    ]
    scratch = [pltpu.VMEM((256 * 2 * D,), jnp.float32)]
    scratch += [pltpu.VMEM((PPW, BPW), jnp.int32)]
    scratch += [pltpu.VMEM((D, BPW), jnp.float32) for _ in range(4)]
    scratch += [pltpu.SemaphoreType.DMA]
    scratch += [pltpu.SemaphoreType.DMA for _ in range(4)]

    def body(tab_hbm, idx_hbm, ok_hbm, ov_hbm, tab_v, idx_v, *refs):
        rows = (refs[0:2], refs[2:4])
        isem = refs[4]
        osems = (refs[5:7], refs[7:9])
        outs = (ok_hbm, ov_hbm)

        wid = lax.axis_index("s") * NC + lax.axis_index("c")
        pb = wid // NBB
        bb = lax.rem(wid, NBB)
        p0 = pb * PPW
        boff = bb * BPW

        pltpu.sync_copy(tab_hbm, tab_v)
        for q in range(PPW):
            pltpu.async_copy(
                idx_hbm.at[pl.ds((p0 + q) * B + boff, BPW)], idx_v.at[q], isem
            )
        for q in range(PPW):
            pltpu.make_async_copy(
                idx_hbm.at[pl.ds(0, BPW)], idx_v.at[q], isem
            ).wait()

        def pair_pair_body(pp, carry):
            for i in range(2):
                q = pp * 2 + i
                p = p0 + q
                l1 = p // L
                l2 = lax.rem(p, L)

                @pl.when(pp > 0)
                def _wait_out():
                    for t in range(2):
                        pltpu.make_async_copy(
                            outs[t].at[0, 0, pl.ds(0, D), pl.ds(0, BPW)],
                            rows[t][i], osems[t][i],
                        ).wait()

                def group_body(g, c2):
                    iv = idx_v[q, pl.ds(g * LANES, LANES)]
                    fb = iv * (2 * D)
                    for d in range(D):
                        ck = (iv + d) & (2 * D - 1)
                        vk = plsc.load_gather(tab_v, [fb + ck])
                        rows[0][i][d, pl.ds(g * LANES, LANES)] = vk
                        cv = (iv + (D + d)) & (2 * D - 1)
                        vv = plsc.load_gather(tab_v, [fb + cv])
                        rows[1][i][d, pl.ds(g * LANES, LANES)] = vv
                    return c2

                lax.fori_loop(0, BPW // LANES, group_body, 0)

                for t in range(2):
                    pltpu.async_copy(
                        rows[t][i],
                        outs[t].at[l1, l2, pl.ds(0, D), pl.ds(boff, BPW)],
                        osems[t][i],
                    )
            return carry

        lax.fori_loop(0, PPW // 2, pair_pair_body, 0)
        for t in range(2):
            for i in range(2):
                pltpu.make_async_copy(
                    outs[t].at[0, 0, pl.ds(0, D), pl.ds(0, BPW)],
                    rows[t][i], osems[t][i],
                ).wait()

    return pl.kernel(
        body, out_type=out_type, mesh=mesh, scratch_types=scratch,
        compiler_params=pltpu.CompilerParams(needs_layout_passes=False))


def kernel(seq_S_u, seq_P_u, T_delta_u, contract_table, time_K_table,
           time_V_table, pop_Q_table, pop_K_table, pop_V_table):
    idx_S = seq_S_u.astype(jnp.int32).reshape(-1)
    idx_Tt = jnp.transpose(T_delta_u.astype(jnp.int32), (1, 2, 0)).reshape(-1)
    idx_P = seq_P_u.astype(jnp.int32).reshape(-1)

    contract_wide = jnp.pad(contract_table, ((0, 0), (0, D)))
    time_kv = jnp.concatenate([time_K_table, time_V_table], axis=1)
    v_ids = jnp.arange(time_kv.shape[0])[:, None]
    j_ids = jnp.arange(2 * D)[None, :]
    time_flat = jnp.take_along_axis(
        time_kv, (j_ids - v_ids) % (2 * D), axis=1).reshape(-1)
    npop = pop_Q_table.shape[0]
    pop_wide = jnp.tile(
        jnp.concatenate(
            [pop_Q_table, pop_K_table, pop_V_table,
             jnp.zeros((npop, D), jnp.float32)], axis=1), (NW, 1))

    (E,) = _wide_gather(contract_wide.shape[0], 2 * D, 1, False)(
        contract_wide, idx_S)
    T_K_t, T_V_t = _time_gather_t()(time_flat, idx_Tt)
    P_Q, P_K, P_V = _wide_gather(npop, 4 * D, 3, False, replicated=True)(
        pop_wide, idx_P)

    T_K = jnp.transpose(T_K_t, (3, 0, 1, 2))
    T_V = jnp.transpose(T_V_t, (3, 0, 1, 2))
    return (E, T_K, T_V, P_Q, P_K, P_V)
